# Initial kernel scaffold; baseline (speedup 1.0000x reference)
import jax
import jax.numpy as jnp
from jax.experimental import pallas as pl


def kernel(x, edge_index, edge_attr, W_s_w, W_s_b, W_r_w, W_r_b, W_e_w, W_e_b, attn_w, attn_b):
    raise NotImplementedError("write your pallas kernel here")


# speedup vs baseline: 1.3618x; 1.3618x over previous
import jax
import jax.numpy as jnp
from jax.experimental import pallas as pl
from jax.experimental.pallas import tpu as pltpu


def _proj_body(x_ref, w_ref, b_ref, o_ref):
    o_ref[...] = (
        jnp.dot(x_ref[...], w_ref[...], preferred_element_type=jnp.float32)
        + b_ref[...]
    )


def _project(x, w, b, blk_m, blk_n):
    M, K = x.shape
    Nc = w.shape[1]
    grid = (M // blk_m, Nc // blk_n)
    return pl.pallas_call(
        _proj_body,
        grid=grid,
        in_specs=[
            pl.BlockSpec((blk_m, K), lambda i, j: (i, 0)),
            pl.BlockSpec((K, blk_n), lambda i, j: (0, j)),
            pl.BlockSpec((1, blk_n), lambda i, j: (0, j)),
        ],
        out_specs=pl.BlockSpec((blk_m, blk_n), lambda i, j: (i, j)),
        out_shape=jax.ShapeDtypeStruct((M, Nc), jnp.float32),
    )(x, w, b)


def kernel(x, edge_index, edge_attr, W_s_w, W_s_b, W_r_w, W_r_b, W_e_w, W_e_b, attn_w, attn_b):
    N, D = x.shape
    E = edge_attr.shape[0]
    H, HD = W_s_b.shape
    HW = H * HD

    senders = edge_index[0]
    receivers = edge_index[1]

    w_sr = jnp.concatenate(
        [W_s_w.reshape(D, HW), W_r_w.reshape(D, HW)], axis=1
    )
    b_sr = jnp.concatenate(
        [W_s_b.reshape(1, HW), W_r_b.reshape(1, HW)], axis=1
    )
    xsr = _project(x, w_sr, b_sr, blk_m=1000, blk_n=256)
    xs = xsr[:, :HW]
    xr = xsr[:, HW:]

    ea = _project(
        edge_attr, W_e_w.reshape(-1, HW), W_e_b.reshape(1, HW),
        blk_m=1000, blk_n=256,
    )

    edges = jnp.take(xs, senders, axis=0) + ea
    att = edges + jnp.take(xr, receivers, axis=0)
    att = jax.nn.leaky_relu(att)
    logits = (
        jnp.einsum('ehk,k->eh', att.reshape(E, H, HD), attn_w.reshape(HD))
        + attn_b[0]
    )
    u = jnp.exp(logits)
    den = jax.ops.segment_sum(u, receivers, num_segments=N)
    weighted = (u[:, :, None] * edges.reshape(E, H, HD)).reshape(E, HW)
    num = jax.ops.segment_sum(weighted, receivers, num_segments=N)
    den_full = jnp.repeat(den, HD, axis=1)
    return jnp.where(den_full > 0, num / den_full, 0.0)


# speedup vs baseline: 1.4272x; 1.0480x over previous
import jax
import jax.numpy as jnp
from jax.experimental import pallas as pl
from jax.experimental.pallas import tpu as pltpu


def _proj_body(x_ref, w_ref, b_ref, o_ref):
    o_ref[...] = (
        jnp.dot(x_ref[...], w_ref[...], preferred_element_type=jnp.float32)
        + b_ref[...]
    )


def _project(x, w, b, blk_m, blk_n):
    M, K = x.shape
    Nc = w.shape[1]
    grid = (M // blk_m, Nc // blk_n)
    return pl.pallas_call(
        _proj_body,
        grid=grid,
        in_specs=[
            pl.BlockSpec((blk_m, K), lambda i, j: (i, 0)),
            pl.BlockSpec((K, blk_n), lambda i, j: (0, j)),
            pl.BlockSpec((1, blk_n), lambda i, j: (0, j)),
        ],
        out_specs=pl.BlockSpec((blk_m, blk_n), lambda i, j: (i, j)),
        out_shape=jax.ShapeDtypeStruct((M, Nc), jnp.float32),
    )(x, w, b)


def kernel(x, edge_index, edge_attr, W_s_w, W_s_b, W_r_w, W_r_b, W_e_w, W_e_b, attn_w, attn_b):
    N, D = x.shape
    E = edge_attr.shape[0]
    H, HD = W_s_b.shape
    HW = H * HD

    senders = edge_index[0]
    receivers = edge_index[1]

    w_sr = jnp.concatenate(
        [W_s_w.reshape(D, HW), W_r_w.reshape(D, HW)], axis=1
    )
    b_sr = jnp.concatenate(
        [W_s_b.reshape(1, HW), W_r_b.reshape(1, HW)], axis=1
    )
    xsr = _project(x, w_sr, b_sr, blk_m=1000, blk_n=256)
    xs = xsr[:, :HW]
    xr = xsr[:, HW:]

    ea = _project(
        edge_attr, W_e_w.reshape(-1, HW), W_e_b.reshape(1, HW),
        blk_m=1000, blk_n=256,
    )

    edges = jnp.take(xs, senders, axis=0) + ea
    att = edges + jnp.take(xr, receivers, axis=0)
    att = jax.nn.leaky_relu(att)
    logits = (
        jnp.einsum('ehk,k->eh', att.reshape(E, H, HD), attn_w.reshape(HD))
        + attn_b[0]
    )
    u = jnp.exp(logits)
    weighted = (u[:, :, None] * edges.reshape(E, H, HD)).reshape(E, HW)
    wu = jnp.concatenate([weighted, u], axis=1)
    seg = jax.ops.segment_sum(wu, receivers, num_segments=N)
    num = seg[:, :HW]
    den_full = jnp.repeat(seg[:, HW:], HD, axis=1)
    return jnp.where(den_full > 0, num / den_full, 0.0)
